# baseline (speedup 1.0000x reference)
import jax
import jax.numpy as jnp
from jax import lax
from jax.experimental import pallas as pl
from jax.experimental.pallas import tpu as pltpu
import numpy as np

B, CIN, S, T = 32, 1, 128, 64
O, N = 1, 2048
STEP, LEAK = 16, 32
KS = STEP + LEAK
PAD = int(np.ceil((KS + STEP) / 2))
FODEP = KS
THETA = 0.1 * (S * CIN)
TOUT = T + 2 * PAD - KS + 1
TP = T + 2 * PAD
MROW = 88
NSH = 8
NQ = KS // NSH

N_TILE = 512
N_BLKS = N // N_TILE


def _builder_body(wt_ref, lab_ref, w3_ref, sup_ref):
    n_blk = pl.program_id(0)
    w = wt_ref[...]
    for q in range(NQ):
        for j in range(NSH):
            kk = float(KS - 1 - (NSH * q + j))
            w3_ref[q, j * S:(j + 1) * S, :] = jnp.maximum(
                0.0, jnp.minimum(kk / STEP, 1.5 * w - kk / LEAK))
    lab = lab_ref[...]
    n_iota = lax.broadcasted_iota(jnp.int32, (B, N_TILE), 1) + n_blk * N_TILE
    sup_ref[:, 0, :] = jnp.where(lab == n_iota, 6.0, 0.0)


def _build_w3(weight, labels):
    wt = weight.T
    lab = labels.reshape(B, 1)
    return pl.pallas_call(
        _builder_body,
        grid=(N_BLKS,),
        in_specs=[
            pl.BlockSpec((S, N_TILE), lambda n: (0, n)),
            pl.BlockSpec((B, 1), lambda n: (0, 0)),
        ],
        out_specs=[
            pl.BlockSpec((NQ, NSH * S, N_TILE), lambda n: (0, 0, n)),
            pl.BlockSpec((B, 1, N_TILE), lambda n: (0, 0, n)),
        ],
        out_shape=[
            jax.ShapeDtypeStruct((NQ, NSH * S, N), jnp.float32),
            jax.ShapeDtypeStruct((B, 1, N), jnp.float32),
        ],
    )(wt, lab)


def _conv_body(xc_ref, w_ref, sup_ref, m_ref, a_ref):
    n_blk = pl.program_id(0)
    acc = jnp.zeros((MROW, N_TILE), jnp.float32)
    for q in range(NQ):
        acc = acc + jnp.dot(
            xc_ref[0, NSH * q:NSH * q + MROW, :], w_ref[q],
            preferred_element_type=jnp.float32)
    acc = acc + sup_ref[...].reshape(1, N_TILE)
    m_ref[0, 0] = jnp.max(acc, axis=1, keepdims=True)
    a_ref[0, 0] = (jnp.argmax(acc, axis=1).astype(jnp.int32)
                   + n_blk * N_TILE)[:, None]


def _conv_max_argmax(xc, w3, sup):
    return pl.pallas_call(
        _conv_body,
        grid=(N_BLKS, B),
        in_specs=[
            pl.BlockSpec((1, TP, NSH * S), lambda n, b: (b, 0, 0)),
            pl.BlockSpec((NQ, NSH * S, N_TILE), lambda n, b: (0, 0, n)),
            pl.BlockSpec((1, 1, N_TILE), lambda n, b: (b, 0, n)),
        ],
        out_specs=[
            pl.BlockSpec((1, 1, MROW, 1), lambda n, b: (n, b, 0, 0)),
            pl.BlockSpec((1, 1, MROW, 1), lambda n, b: (n, b, 0, 0)),
        ],
        out_shape=[
            jax.ShapeDtypeStruct((N_BLKS, B, MROW, 1), jnp.float32),
            jax.ShapeDtypeStruct((N_BLKS, B, MROW, 1), jnp.int32),
        ],
    )(xc, w3, sup)


def _onehot_body(m_ref, a_ref, out_ref):
    best = m_ref[0, 0]
    bi = a_ref[0, 0]
    for p in range(1, N_BLKS):
        mk = m_ref[p, 0]
        ak = a_ref[p, 0]
        upd = mk > best
        best = jnp.where(upd, mk, best)
        bi = jnp.where(upd, ak, bi)
    tio = lax.broadcasted_iota(jnp.int32, (MROW, 1), 0)
    valid = tio < TOUT
    q = jnp.logical_and(valid, best > jnp.float32(THETA))
    big = jnp.int32(4 * MROW)
    t1 = jnp.min(jnp.where(q, tio, big))
    q2 = jnp.logical_and(q, tio >= t1 + FODEP)
    t2 = jnp.min(jnp.where(q2, tio, big))
    a1 = jnp.sum(jnp.where(tio == t1, bi, 0))
    a2 = jnp.sum(jnp.where(tio == t2, bi, 0))
    n_io = lax.broadcasted_iota(jnp.int32, (N, TOUT), 0)
    t_io = lax.broadcasted_iota(jnp.int32, (N, TOUT), 1)
    hit1 = jnp.logical_and(n_io == a1, t_io == t1)
    hit2 = jnp.logical_and(n_io == a2, t_io == t2)
    out_ref[0, 0] = jnp.where(jnp.logical_or(hit1, hit2), 1.0, 0.0)


def _onehot(mpart, apart):
    return pl.pallas_call(
        _onehot_body,
        grid=(B,),
        in_specs=[
            pl.BlockSpec((N_BLKS, 1, MROW, 1), lambda b: (0, b, 0, 0)),
            pl.BlockSpec((N_BLKS, 1, MROW, 1), lambda b: (0, b, 0, 0)),
        ],
        out_specs=pl.BlockSpec((1, 1, N, TOUT), lambda b: (b, 0, 0, 0)),
        out_shape=jax.ShapeDtypeStruct((B, O, N, TOUT), jnp.float32),
    )(mpart, apart)


def _prep_body(xt_ref, xc_ref):
    for j in range(NSH):
        xc_ref[0, :, j * S:(j + 1) * S] = xt_ref[0, j:j + TP, :]


def _build_xc(input_spikes):
    x = input_spikes.reshape(B, S, T)
    xp = jnp.pad(x, ((0, 0), (0, 0), (PAD, PAD)))
    xt = jnp.transpose(xp, (0, 2, 1))
    xt = jnp.pad(xt, ((0, 0), (0, NSH), (0, 0)))
    return pl.pallas_call(
        _prep_body,
        grid=(B,),
        in_specs=[pl.BlockSpec((1, TP + NSH, S), lambda b: (b, 0, 0))],
        out_specs=pl.BlockSpec((1, TP, NSH * S), lambda b: (b, 0, 0)),
        out_shape=jax.ShapeDtypeStruct((B, TP, NSH * S), jnp.float32),
    )(xt)


@jax.jit
def kernel(input_spikes, weight, labels):
    xc = _build_xc(input_spikes)
    w3, sup = _build_w3(weight, labels.astype(jnp.int32))
    mpart, apart = _conv_max_argmax(xc, w3, sup)
    return _onehot(mpart, apart)
